# Initial kernel scaffold; baseline (speedup 1.0000x reference)
import jax
import jax.numpy as jnp
from jax.experimental import pallas as pl


def kernel(field, grid_points, sample_positions):
    raise NotImplementedError("write your pallas kernel here")


# speedup vs baseline: 1266.4824x; 1266.4824x over previous
import functools

import jax
import jax.numpy as jnp
from jax import lax
from jax.experimental import pallas as pl
from jax.experimental.pallas import tpu as pltpu, tpu_sc as plsc

NC, NS, L = 2, 16, 16
NW = NC * NS
B, G, D, N = 8, 8192, 64, 65536
SAMPLES_PER_W = (B * N) // NW
W_PER_BATCH = N // SAMPLES_PER_W
CHUNK = 256
NROW = CHUNK // 128
NCHUNKS = SAMPLES_PER_W // CHUNK
NVEC = CHUNK // L


def _body(field_hbm, grid_hbm, pos_hbm, out_hbm,
          grid_v, pos_v, idxl_v, idxr_v, w_v, rows_l, rows_r, out_v, sem):
    wid = lax.axis_index("s") * NC + lax.axis_index("c")
    b = wid // W_PER_BATCH
    gbase = wid * SAMPLES_PER_W

    pltpu.sync_copy(grid_hbm.at[pl.ds(b * G, G)], grid_v)

    row_base = jnp.full((L,), b * G, jnp.int32)

    def chunk_body(c, _):
        base = gbase + c * CHUNK
        pltpu.sync_copy(pos_hbm.at[pl.ds(base, CHUNK)], pos_v)

        def search_body(i, _):
            pos = pos_v[pl.ds(i * L, L)]
            idx = (pos * 0.0).astype(jnp.int32)
            bit = G // 2
            while bit > 0:
                j = idx + bit
                g = plsc.load_gather(grid_v, [j])
                idx = jnp.where(g <= pos, j, idx)
                bit //= 2
            idx = jnp.minimum(idx, G - 2)
            gl = plsc.load_gather(grid_v, [idx])
            gr = plsc.load_gather(grid_v, [idx + 1])
            wr = jnp.clip((pos - gl) / jnp.maximum(gr - gl, 1e-8), 0.0, 1.0)
            r = i // (128 // L)
            col = (i % (128 // L)) * L
            idxl_v[r, pl.ds(col, L)] = row_base + idx
            idxr_v[r, pl.ds(col, L)] = row_base + idx + 1
            w_v[pl.ds(i * L, L)] = wr
            return 0

        lax.fori_loop(0, NVEC, search_body, 0)

        copies = []
        for r in range(NROW):
            copies.append(pltpu.async_copy(
                field_hbm.at[idxl_v.at[r]],
                rows_l.at[pl.ds(r * 128, 128)], sem))
            copies.append(pltpu.async_copy(
                field_hbm.at[idxr_v.at[r]],
                rows_r.at[pl.ds(r * 128, 128)], sem))
        for cp in copies:
            cp.wait()

        def lerp_body(n, _):
            wrv = plsc.load_gather(w_v, [jnp.full((L,), n, jnp.int32)])
            wlv = 1.0 - wrv
            for d in range(D // L):
                fl = rows_l[n, pl.ds(d * L, L)]
                fr = rows_r[n, pl.ds(d * L, L)]
                out_v[n, pl.ds(d * L, L)] = wlv * fl + wrv * fr
            return 0

        lax.fori_loop(0, CHUNK, lerp_body, 0)

        pltpu.sync_copy(out_v, out_hbm.at[pl.ds(base, CHUNK)])
        return 0

    lax.fori_loop(0, NCHUNKS, chunk_body, 0)


@jax.jit
def kernel(field, grid_points, sample_positions):
    field2 = field.reshape(B * G, D)
    grid_flat = grid_points.reshape(B * G)
    pos_flat = sample_positions.reshape(B * N)

    mesh = plsc.VectorSubcoreMesh(
        core_axis_name="c", subcore_axis_name="s",
        num_cores=NC, num_subcores=NS)
    out = pl.kernel(
        _body,
        out_type=jax.ShapeDtypeStruct((B * N, D), jnp.float32),
        mesh=mesh,
        scratch_types=[
            pltpu.VMEM((G,), jnp.float32),
            pltpu.VMEM((CHUNK,), jnp.float32),
            pltpu.VMEM((NROW, 128), jnp.int32),
            pltpu.VMEM((NROW, 128), jnp.int32),
            pltpu.VMEM((CHUNK,), jnp.float32),
            pltpu.VMEM((CHUNK, D), jnp.float32),
            pltpu.VMEM((CHUNK, D), jnp.float32),
            pltpu.VMEM((CHUNK, D), jnp.float32),
            pltpu.SemaphoreType.DMA,
        ],
        compiler_params=pltpu.CompilerParams(
            needs_layout_passes=False, use_tc_tiling_on_sc=False),
    )(field2, grid_flat, pos_flat)
    return out.reshape(B, N, D)
